# Initial kernel scaffold; baseline (speedup 1.0000x reference)
import jax
import jax.numpy as jnp
from jax.experimental import pallas as pl


def kernel(x, iM1, iM2, iM3, iM4, CiM1, CiM2, CiM3, CiM4, W):
    raise NotImplementedError("write your pallas kernel here")


# speedup vs baseline: 16.2612x; 16.2612x over previous
import dataclasses
import functools

import jax
import jax.numpy as jnp
from jax import lax
from jax.experimental import pallas as pl
from jax.experimental.pallas import tpu as pltpu
from jax.experimental.pallas import tpu_sc as plsc

DIMENSIONS = 10000
NUM_LEVELS = 21
N_GRAM_SIZE = 4
WINDOW = 256
NUM_CLASSES = 5
BATCH = 256
STRIDE = WINDOW // 5
N_SAMPLES = (WINDOW + STRIDE - 1) // STRIDE
N_ROWS = 4 * N_SAMPLES
LANES = 16
ROW_PAD = 24


def _sc_hist_body(x_hbm, part_hbm, xrow_v, part_v):
    w = lax.axis_index("s") * 2 + lax.axis_index("c")

    @pl.when(w < N_ROWS)
    def _():
        pltpu.sync_copy(x_hbm.at[w], xrow_v)
        zeros16 = jnp.zeros((LANES,), jnp.float32)
        for l in range(LANES):
            part_v[l, pl.ds(0, LANES)] = zeros16
            part_v[l, pl.ds(ROW_PAD - LANES, LANES)] = zeros16
        lane = lax.iota(jnp.int32, LANES)
        ones = jnp.ones((LANES,), jnp.float32)
        for k in range(BATCH // LANES):
            v = xrow_v[pl.ds(k * LANES, LANES)]
            plsc.addupdate_scatter(part_v, [lane, v], ones)
        pltpu.sync_copy(part_v, part_hbm.at[w])


def _roll1(a):
    return jnp.concatenate([a[:, DIMENSIONS - 1:], a[:, :DIMENSIONS - 1]], axis=1)


def _tc_body(part, iM1, iM2, iM3, iM4, CiM1, CiM2, CiM3, CiM4, W, out_ref):
    iM_refs = (iM1, iM2, iM3, iM4)
    CiM_refs = (CiM1, CiM2, CiM3, CiM4)

    counts24 = jnp.sum(part[...], axis=1)
    shv = None
    for c in range(4):
        T_c = iM_refs[c][:NUM_LEVELS, :] * CiM_refs[c][...]
        counts_c = counts24[c * N_SAMPLES:(c + 1) * N_SAMPLES, :NUM_LEVELS]
        part_mm = jax.lax.dot_general(
            counts_c, T_c, (((1,), (0,)), ((), ())),
            preferred_element_type=jnp.float32)
        shv = part_mm if shv is None else shv + part_mm

    rows = [shv[i:i + 1, :] for i in range(N_SAMPLES)]
    A = jnp.zeros_like(rows[0])
    for i in range(N_SAMPLES - N_GRAM_SIZE + 1):
        g = rows[i] + A
        for n in range(1, N_GRAM_SIZE):
            g = _roll1(g) * (rows[i + n] + A)
        A = A + g
    total = rows[0]
    for i in range(1, N_SAMPLES):
        total = total + rows[i]
    total = total + float(N_SAMPLES) * A

    enc = jnp.where(total > 0, 1.0, -1.0)
    out_ref[...] = jax.lax.dot_general(
        enc, W[...], (((1,), (1,)), ((), ())),
        preferred_element_type=jnp.float32)


@jax.jit
def _run(x, iM1, iM2, iM3, iM4, CiM1, CiM2, CiM3, CiM4, W):
    xs = x[:, ::STRIDE, :]
    xs24 = jnp.transpose(xs, (2, 1, 0)).reshape(N_ROWS, BATCH)

    cp = pltpu.CompilerParams()
    if "needs_layout_passes" in pltpu.CompilerParams.__dataclass_fields__:
        cp = dataclasses.replace(cp, needs_layout_passes=False)
    hist = pl.kernel(
        _sc_hist_body,
        compiler_params=cp,
        out_type=jax.ShapeDtypeStruct((N_ROWS, LANES, ROW_PAD), jnp.float32),
        mesh=plsc.VectorSubcoreMesh(core_axis_name="c", subcore_axis_name="s"),
        scratch_types=[
            pltpu.VMEM((BATCH,), jnp.int32),
            pltpu.VMEM((LANES, ROW_PAD), jnp.float32),
        ],
    )
    part = hist(xs24)

    full = lambda a: pl.BlockSpec(a.shape, lambda i: (0,) * a.ndim)
    iM_spec = pl.BlockSpec((ROW_PAD, DIMENSIONS), lambda i: (0, 0))
    out = pl.pallas_call(
        _tc_body,
        grid=(1,),
        out_shape=jax.ShapeDtypeStruct((1, NUM_CLASSES), jnp.float32),
        in_specs=[full(part), iM_spec, iM_spec, iM_spec, iM_spec,
                  full(CiM1), full(CiM2), full(CiM3), full(CiM4), full(W)],
        out_specs=pl.BlockSpec((1, NUM_CLASSES), lambda i: (0, 0)),
    )(part, iM1, iM2, iM3, iM4, CiM1, CiM2, CiM3, CiM4, W)
    return out.reshape(NUM_CLASSES)


def kernel(x, iM1, iM2, iM3, iM4, CiM1, CiM2, CiM3, CiM4, W):
    return _run(x, iM1, iM2, iM3, iM4, CiM1, CiM2, CiM3, CiM4, W)


# speedup vs baseline: 20.4643x; 1.2585x over previous
import functools

import jax
import jax.numpy as jnp
from jax.experimental import pallas as pl

DIMENSIONS = 10000
NUM_LEVELS = 21
N_GRAM_SIZE = 4
WINDOW = 256
NUM_CLASSES = 5
BATCH = 256
STRIDE = WINDOW // 5
N_SAMPLES = (WINDOW + STRIDE - 1) // STRIDE


def _roll1(a):
    return jnp.concatenate([a[:, DIMENSIONS - 1:], a[:, :DIMENSIONS - 1]], axis=1)


def _hd_kernel(xs1, xs2, xs3, xs4, iM1, iM2, iM3, iM4,
               CiM1, CiM2, CiM3, CiM4, W, out_ref):
    xs_refs = (xs1, xs2, xs3, xs4)
    iM_refs = (iM1, iM2, iM3, iM4)
    CiM_refs = (CiM1, CiM2, CiM3, CiM4)

    shv = None
    for c in range(4):
        T_c = iM_refs[c][:NUM_LEVELS, :] * CiM_refs[c][...]
        xc = xs_refs[c][...]
        cols = [
            jnp.sum((xc == v).astype(jnp.float32), axis=1, keepdims=True)
            for v in range(NUM_LEVELS)
        ]
        counts = jnp.concatenate(cols, axis=1)
        part = jax.lax.dot_general(
            counts, T_c, (((1,), (0,)), ((), ())),
            preferred_element_type=jnp.float32)
        shv = part if shv is None else shv + part

    rows = [shv[i:i + 1, :] for i in range(N_SAMPLES)]
    A = jnp.zeros_like(rows[0])
    for i in range(N_SAMPLES - N_GRAM_SIZE + 1):
        g = rows[i] + A
        for n in range(1, N_GRAM_SIZE):
            g = _roll1(g) * (rows[i + n] + A)
        A = A + g
    total = rows[0]
    for i in range(1, N_SAMPLES):
        total = total + rows[i]
    total = total + float(N_SAMPLES) * A

    enc = jnp.where(total > 0, 1.0, -1.0)
    out_ref[...] = jax.lax.dot_general(
        enc, W[...], (((1,), (1,)), ((), ())),
        preferred_element_type=jnp.float32)


@functools.partial(jax.jit, static_argnames=())
def _run(x, iM1, iM2, iM3, iM4, CiM1, CiM2, CiM3, CiM4, W):
    xs = x[:, ::STRIDE, :]
    xs_t = jnp.transpose(xs, (2, 1, 0))
    xs_list = [xs_t[c] for c in range(4)]
    full = lambda a: pl.BlockSpec(a.shape, lambda i: (0,) * a.ndim)
    iM_spec = pl.BlockSpec((24, DIMENSIONS), lambda i: (0, 0))
    out = pl.pallas_call(
        _hd_kernel,
        grid=(1,),
        out_shape=jax.ShapeDtypeStruct((1, NUM_CLASSES), jnp.float32),
        in_specs=[full(xs_list[0]), full(xs_list[1]), full(xs_list[2]),
                  full(xs_list[3]), iM_spec, iM_spec, iM_spec, iM_spec,
                  full(CiM1), full(CiM2), full(CiM3), full(CiM4), full(W)],
        out_specs=pl.BlockSpec((1, NUM_CLASSES), lambda i: (0, 0)),
    )(*xs_list, iM1, iM2, iM3, iM4, CiM1, CiM2, CiM3, CiM4, W)
    return out.reshape(NUM_CLASSES)


def kernel(x, iM1, iM2, iM3, iM4, CiM1, CiM2, CiM3, CiM4, W):
    return _run(x, iM1, iM2, iM3, iM4, CiM1, CiM2, CiM3, CiM4, W)
